# baseline (speedup 1.0000x reference)
import functools

import jax
import jax.numpy as jnp
from jax import lax
from jax.experimental import pallas as pl
from jax.experimental.pallas import tpu as pltpu, tpu_sc as plsc

NUM_STREAMS = 8
B = 128
L = 32768
LANES = 16
NBUF = 3
CB = 512
SC_STREAMS = (5, 6, 7)
TC_NSTREAMS = 5

_info = plsc.get_sparse_core_info()
NC, NS = _info.num_cores, _info.num_subcores
NW = NC * NS
ROWS_PER_WORKER = B // NW

_STRIDES = [(s + 1) * 10 for s in range(NUM_STREAMS)]
_COUNTS = [(L + st - 1) // st for st in _STRIDES]
_CHUNKS = [(c + LANES - 1) // LANES for c in _COUNTS]


def _positions(c, stream_idx):
    iota = lax.iota(jnp.int32, LANES)
    nums = jnp.minimum(c * LANES + iota, _COUNTS[stream_idx] - 1)
    return nums * _STRIDES[stream_idx]


def _fix(buf, save, add_v, stream_idx):
    def body(c, carry):
        pos = _positions(c, stream_idx)
        g = plsc.load_gather(buf, [pos])
        save[pl.ds(c * LANES, LANES)] = g
        y = g + add_v
        y = jnp.where(y >= 4096.0, y - 4096.0, y)
        plsc.store_scatter(buf, [pos], y)
        return carry

    lax.fori_loop(0, _CHUNKS[stream_idx], body, 0)


def _revert(buf, save, stream_idx):
    def body(c, carry):
        pos = _positions(c, stream_idx)
        plsc.store_scatter(buf, [pos], save[pl.ds(c * LANES, LANES)])
        return carry

    lax.fori_loop(0, _CHUNKS[stream_idx], body, 0)


def _make_sc_kernel():
    mesh = plsc.VectorSubcoreMesh(core_axis_name="c", subcore_axis_name="s")

    scratch = [
        pltpu.VMEM((L,), jnp.float32),
        pltpu.VMEM((L,), jnp.float32),
        pltpu.VMEM((L,), jnp.float32),
        pltpu.VMEM((NUM_STREAMS * LANES,), jnp.float32),
    ]
    for _ in range(2):
        for s in SC_STREAMS:
            scratch.append(pltpu.VMEM((_CHUNKS[s] * LANES,), jnp.float32))
    scratch.extend([pltpu.SemaphoreType.DMA] * (2 * NBUF))

    nsub = len(SC_STREAMS)

    @functools.partial(
        pl.kernel,
        mesh=mesh,
        compiler_params=pltpu.CompilerParams(
            needs_layout_passes=False, use_tc_tiling_on_sc=False),
        out_type=jax.ShapeDtypeStruct((NUM_STREAMS * B, L), jnp.float32),
        scratch_types=scratch,
    )
    def sc_kernel(base_hbm, adds_hbm, out_hbm, buf0, buf1, buf2, adds_v,
                  *rest):
        bufs = (buf0, buf1, buf2)
        saves = [rest[nsub * lane:nsub * lane + nsub] for lane in range(2)]
        sems = rest[2 * nsub:]
        sem_in = sems[:NBUF]
        sem_out = sems[NBUF:]

        wid = lax.axis_index("c") * NS + lax.axis_index("s")
        row0 = wid * ROWS_PER_WORKER

        pltpu.sync_copy(adds_hbm, adds_v)

        ind = [None] * ROWS_PER_WORKER
        outd = {}

        def start_in(r):
            ind[r] = pltpu.async_copy(
                base_hbm.at[row0 + r], bufs[r % NBUF], sem_in[r % NBUF])

        def step(r, j):
            s = SC_STREAMS[j]
            if j == 0:
                ind[r].wait()
            else:
                outd[(r, j - 1)].wait()
                _revert(bufs[r % NBUF], saves[r % 2][j - 1],
                        SC_STREAMS[j - 1])
            _fix(bufs[r % NBUF], saves[r % 2][j],
                 adds_v[pl.ds(s * LANES, LANES)], s)
            outd[(r, j)] = pltpu.async_copy(
                bufs[r % NBUF], out_hbm.at[s * B + row0 + r],
                sem_out[r % NBUF])

        start_in(0)
        start_in(1)
        start_in(2)
        for j in range(nsub):
            step(0, j)
            step(1, j)
        outd[(0, nsub - 1)].wait()
        start_in(3)
        for j in range(nsub):
            step(2, j)
            step(3, j)
        outd[(1, nsub - 1)].wait()
        outd[(2, nsub - 1)].wait()
        outd[(3, nsub - 1)].wait()

    return sc_kernel


_sc_kernel = _make_sc_kernel()


def _tc_body(adds_ref, base_ref, alias_ref, out_ref):
    del alias_ref
    c = pl.program_id(1)
    s = pl.program_id(0)
    x = base_ref[...]
    cols = c * CB + lax.broadcasted_iota(jnp.int32, (B, CB), 1)
    st = (s + 1) * 10
    m = (cols % st) == 0
    a = adds_ref[s, 0]
    y = x + a
    y = jnp.where(y >= 4096.0, y - 4096.0, y)
    out_ref[...] = jnp.where(m, y, x)


def _tc_call(adds, base, first):
    return pl.pallas_call(
        _tc_body,
        grid=(TC_NSTREAMS, L // CB),
        in_specs=[
            pl.BlockSpec((NUM_STREAMS, 128), lambda s, c: (0, 0)),
            pl.BlockSpec((B, CB), lambda s, c: (0, c)),
            pl.BlockSpec((8, 128), lambda s, c: (0, 0)),
        ],
        out_specs=pl.BlockSpec((B, CB), lambda s, c: (s, c)),
        out_shape=jax.ShapeDtypeStruct((NUM_STREAMS * B, L), jnp.float32),
        input_output_aliases={2: 0},
        compiler_params=pltpu.CompilerParams(
            dimension_semantics=("arbitrary", "arbitrary")),
    )(adds, base, first)


def kernel(base_inputs, current_step):
    active = (jnp.asarray(current_step) > 0).astype(jnp.float32)
    adds_flat = (jnp.arange(NUM_STREAMS, dtype=jnp.float32)[:, None] * active
                 * jnp.ones((1, LANES), jnp.float32)).reshape(-1)
    adds_tc = (jnp.arange(NUM_STREAMS, dtype=jnp.float32)[:, None] * active
               * jnp.ones((1, 128), jnp.float32))
    first = _sc_kernel(base_inputs, adds_flat)
    return _tc_call(adds_tc, base_inputs, first)


# speedup vs baseline: 1.8428x; 1.8428x over previous
import functools

import jax
import jax.numpy as jnp
from jax import lax
from jax.experimental import pallas as pl
from jax.experimental.pallas import tpu as pltpu, tpu_sc as plsc

NUM_STREAMS = 8
B = 128
L = 32768
LANES = 16
NBUF = 3

_info = plsc.get_sparse_core_info()
NC, NS = _info.num_cores, _info.num_subcores
NW = NC * NS
ROWS_PER_WORKER = B // NW

_STRIDES = [(s + 1) * 10 for s in range(NUM_STREAMS)]
_COUNTS = [(L + st - 1) // st for st in _STRIDES]
_CHUNKS = [(c + LANES - 1) // LANES for c in _COUNTS]


def _positions(c, stream_idx):
    iota = lax.iota(jnp.int32, LANES)
    nums = jnp.minimum(c * LANES + iota, _COUNTS[stream_idx] - 1)
    return nums * _STRIDES[stream_idx]


def _fix(buf, save, add_v, stream_idx):
    def body(c, carry):
        pos = _positions(c, stream_idx)
        g = plsc.load_gather(buf, [pos])
        save[pl.ds(c * LANES, LANES)] = g
        y = g + add_v
        y = jnp.where(y >= 4096.0, y - 4096.0, y)
        plsc.store_scatter(buf, [pos], y)
        return carry

    lax.fori_loop(0, _CHUNKS[stream_idx], body, 0)


def _revert(buf, save, stream_idx):
    def body(c, carry):
        pos = _positions(c, stream_idx)
        plsc.store_scatter(buf, [pos], save[pl.ds(c * LANES, LANES)])
        return carry

    lax.fori_loop(0, _CHUNKS[stream_idx], body, 0)


def _make_sc_kernel():
    mesh = plsc.VectorSubcoreMesh(core_axis_name="c", subcore_axis_name="s")

    scratch = [
        pltpu.VMEM((L,), jnp.float32),
        pltpu.VMEM((L,), jnp.float32),
        pltpu.VMEM((L,), jnp.float32),
        pltpu.VMEM((NUM_STREAMS * LANES,), jnp.float32),
    ]
    for _ in range(2):
        for s in range(1, NUM_STREAMS):
            scratch.append(pltpu.VMEM((_CHUNKS[s] * LANES,), jnp.float32))
    scratch.extend([pltpu.SemaphoreType.DMA] * (2 * NBUF))

    @functools.partial(
        pl.kernel,
        mesh=mesh,
        compiler_params=pltpu.CompilerParams(
            needs_layout_passes=False, use_tc_tiling_on_sc=False),
        out_type=jax.ShapeDtypeStruct((NUM_STREAMS * B, L), jnp.float32),
        scratch_types=scratch,
    )
    def sc_kernel(base_hbm, adds_hbm, out_hbm, buf0, buf1, buf2, adds_v,
                  *rest):
        bufs = (buf0, buf1, buf2)
        saves = [rest[7 * lane:7 * lane + 7] for lane in range(2)]
        sems = rest[14:]
        sem_in = sems[:NBUF]
        sem_out = sems[NBUF:]

        wid = lax.axis_index("c") * NS + lax.axis_index("s")
        row0 = wid * ROWS_PER_WORKER

        pltpu.sync_copy(adds_hbm, adds_v)

        ind = [None] * ROWS_PER_WORKER
        outd = {}

        def start_in(r):
            ind[r] = pltpu.async_copy(
                base_hbm.at[row0 + r], bufs[r % NBUF], sem_in[r % NBUF])

        def step(r, s):
            if s == 0:
                ind[r].wait()
            else:
                outd[(r, s - 1)].wait()
                if s - 1 >= 1:
                    _revert(bufs[r % NBUF], saves[r % 2][s - 2], s - 1)
                _fix(bufs[r % NBUF], saves[r % 2][s - 1],
                     adds_v[pl.ds(s * LANES, LANES)], s)
            outd[(r, s)] = pltpu.async_copy(
                bufs[r % NBUF], out_hbm.at[s * B + row0 + r],
                sem_out[r % NBUF])

        start_in(0)
        start_in(1)
        start_in(2)
        for s in range(NUM_STREAMS):
            step(0, s)
            step(1, s)
        outd[(0, NUM_STREAMS - 1)].wait()
        start_in(3)
        for s in range(NUM_STREAMS):
            step(2, s)
            step(3, s)
        outd[(1, NUM_STREAMS - 1)].wait()
        outd[(2, NUM_STREAMS - 1)].wait()
        outd[(3, NUM_STREAMS - 1)].wait()

    return sc_kernel


_sc_kernel = _make_sc_kernel()


def kernel(base_inputs, current_step):
    active = (jnp.asarray(current_step) > 0).astype(jnp.float32)
    adds = (jnp.arange(NUM_STREAMS, dtype=jnp.float32)[:, None] * active
            * jnp.ones((1, LANES), jnp.float32)).reshape(-1)
    return _sc_kernel(base_inputs, adds)
